# baseline (speedup 1.0000x reference)
import functools
import jax
import jax.numpy as jnp
from jax import lax
from jax.experimental import pallas as pl
from jax.experimental.pallas import tpu as pltpu

HEADS = 3
EMBED = 1024
NUM_GRAPHS = 8
RATIO = 0.7


def _mm_body(a_ref, b_ref, o_ref):
    o_ref[...] = jnp.dot(a_ref[...], b_ref[...],
                         preferred_element_type=jnp.float32)


def _mm(a, b, bm=512, bn=512):
    M, K = a.shape
    K2, N = b.shape
    assert K == K2
    Mp = (M + bm - 1) // bm * bm
    Np = (N + bn - 1) // bn * bn
    if Mp != M:
        a = jnp.pad(a, ((0, Mp - M), (0, 0)))
    if Np != N:
        b = jnp.pad(b, ((0, 0), (0, Np - N)))
    out = pl.pallas_call(
        _mm_body,
        grid=(Mp // bm, Np // bn),
        in_specs=[pl.BlockSpec((bm, K), lambda i, j: (i, 0)),
                  pl.BlockSpec((K, bn), lambda i, j: (0, j))],
        out_specs=pl.BlockSpec((bm, bn), lambda i, j: (i, j)),
        out_shape=jax.ShapeDtypeStruct((Mp, Np), jnp.float32),
    )(a, b)
    return out[:M, :N]


def kernel(x, edge_att, edge_index, batch_index, W1, as1, ad1, b1, Wl1, bl1, p1,
           W2, as2, ad2, b2, Wl2, bl2, p2, W3, as3, ad3, b3, Wl3, bl3, p3,
           Wf1, bf1, Wf2, bf2):
    p = dict(W1=W1, as1=as1, ad1=ad1, b1=b1, Wl1=Wl1, bl1=bl1, p1=p1,
             W2=W2, as2=as2, ad2=ad2, b2=b2, Wl2=Wl2, bl2=bl2, p2=p2,
             W3=W3, as3=as3, ad3=ad3, b3=b3, Wl3=Wl3, bl3=bl3, p3=p3)
    N = x.shape[0]
    E = edge_index.shape[1]
    src0, dst0 = edge_index[0], edge_index[1]
    loop = jnp.arange(N, dtype=edge_index.dtype)
    src = jnp.concatenate([src0, loop])
    dst = jnp.concatenate([dst0, loop])
    batch = batch_index

    nmask = jnp.ones((N,), bool)
    emask = jnp.ones((E,), bool)
    h = x
    reps = []
    for i in (1, 2, 3):
        W = p['W%d' % i]
        a_s = p['as%d' % i]
        a_d = p['ad%d' % i]
        b = p['b%d' % i]
        Wl = p['Wl%d' % i]
        bl = p['bl%d' % i]
        Fin = W.shape[0]
        Wr = W.reshape(Fin, HEADS, EMBED)
        As = jnp.einsum('fhe,he->fh', Wr, a_s)
        Ad = jnp.einsum('fhe,he->fh', Wr, a_d)
        Wlr = Wl.reshape(HEADS, EMBED, EMBED)
        C = jnp.einsum('fhe,heg->hfg', Wr, Wlr)
        Cc = C.transpose(1, 0, 2).reshape(Fin * HEADS, EMBED)

        asn = h @ As
        adn = h @ Ad
        valid = jnp.concatenate([emask, nmask])
        logits = jax.nn.leaky_relu(asn[src] + adn[dst], 0.2)
        logits = jnp.where(valid[:, None], logits, -jnp.inf)
        m = jax.ops.segment_max(logits, dst, num_segments=N)
        m = jnp.where(jnp.isfinite(m), m, 0.0)
        e = jnp.exp(logits - m[dst])
        ssum = jax.ops.segment_sum(e, dst, num_segments=N)
        alpha = e / (ssum[dst] + 1e-16)

        agg = jax.ops.segment_sum(alpha[:, :, None] * h[src][:, None, :], dst,
                                  num_segments=N)
        aggc = agg.transpose(0, 2, 1).reshape(N, Fin * HEADS)
        hn = _mm(aggc, Cc) + (b @ Wl + bl)[None, :]

        w = p['p%d' % i]
        score = jnp.tanh((hn @ w) / (jnp.linalg.norm(w) + 1e-16))
        s = jnp.where(nmask, score, -jnp.inf)
        same = batch[:, None] == batch[None, :]
        gt = (s[None, :] > s[:, None]) | ((s[None, :] == s[:, None]) & (loop[None, :] < loop[:, None]))
        rank = jnp.sum((same & gt & nmask[None, :]).astype(jnp.int32), axis=1)
        counts = jax.ops.segment_sum(nmask.astype(jnp.int32), batch, num_segments=NUM_GRAPHS)
        kper = jnp.ceil(RATIO * counts.astype(jnp.float32)).astype(jnp.int32)
        kept = nmask & (rank < kper[batch])
        hsc = jnp.where(kept[:, None], hn * score[:, None], 0.0)
        emask = emask & kept[src0] & kept[dst0]
        nmask = kept
        cnt = jax.ops.segment_sum(kept.astype(jnp.int32), batch, num_segments=NUM_GRAPHS)
        mean = jax.ops.segment_sum(hsc, batch, num_segments=NUM_GRAPHS) / jnp.maximum(cnt, 1).astype(x.dtype)[:, None]
        hmax = jnp.where(kept[:, None], hsc, -jnp.inf)
        mx = jax.ops.segment_max(hmax, batch, num_segments=NUM_GRAPHS)
        mx = jnp.where(jnp.isfinite(mx), mx, 0.0)
        reps.append(jnp.concatenate([mean, mx], axis=-1))
        h = hsc

    z = reps[0] + reps[1] + reps[2]
    z = jnp.maximum(z @ Wf1 + bf1, 0.0)
    z = z @ Wf2 + bf2
    return z


# speedup vs baseline: 5.3240x; 5.3240x over previous
import functools
import jax
import jax.numpy as jnp
from jax import lax
from jax.experimental import pallas as pl
from jax.experimental.pallas import tpu as pltpu
from jax.experimental.pallas import tpu_sc as plsc

HEADS = 3
EMBED = 1024
NUM_GRAPHS = 8
RATIO = 0.7

N_NODES = 10000
N_EDGES = 160000
NPAD = 10240
WIN = 64
NWIN = NPAD // WIN
CH = 256
NE = N_EDGES + N_NODES
NCHUNK = (NE + CH - 1) // CH + NWIN
EPP = NCHUNK * CH
BM = 512

NEG = -1e30


def _mm_body(a_ref, b_ref, o_ref):
    o_ref[...] = jnp.dot(a_ref[...], b_ref[...],
                         preferred_element_type=jnp.float32)


def _mm(a, b):
    (M, K), (_, N) = a.shape, b.shape
    return pl.pallas_call(
        _mm_body,
        grid=(1,),
        in_specs=[pl.BlockSpec((M, K), lambda i: (0, 0)),
                  pl.BlockSpec((K, N), lambda i: (0, 0))],
        out_specs=pl.BlockSpec((M, N), lambda i: (0, 0)),
        out_shape=jax.ShapeDtypeStruct((M, N), jnp.float32),
    )(a, b)


def _attn_body(h_ref, aa_ref, kept_ref, o_ref):
    i0 = pl.program_id(0) * BM
    r = jnp.dot(h_ref[...], aa_ref[...], preferred_element_type=jnp.float32)
    k = kept_ref[0, pl.ds(i0, BM)]
    asn = r[:, 0:3]
    adn = r[:, 3:6]
    drop = (1.0 - k)[:, None]
    sl = asn + adn
    cself = jnp.where(sl > 0, sl, 0.2 * sl) + drop * 1e30
    asnm = asn + drop * NEG
    out = jnp.concatenate(
        [asnm, adn, cself, jnp.zeros((BM, 7), jnp.float32)], axis=1)
    o_ref[...] = out.T


def _attn_prep(h, AsAd, keptf):
    return pl.pallas_call(
        _attn_body,
        grid=(NPAD // BM,),
        in_specs=[pl.BlockSpec((BM, h.shape[1]), lambda i: (i, 0)),
                  pl.BlockSpec((h.shape[1], 128), lambda i: (0, 0)),
                  pl.BlockSpec((1, NPAD), lambda i: (0, 0))],
        out_specs=pl.BlockSpec((16, BM), lambda i: (0, i)),
        out_shape=jax.ShapeDtypeStruct((16, NPAD), jnp.float32),
    )(h, AsAd, keptf)


def _sc_meta(att9, srcp, dstp, dstwf):
    mesh = plsc.VectorSubcoreMesh(core_axis_name="c", subcore_axis_name="s")
    cpw = (NCHUNK + 31) // 32

    @functools.partial(
        pl.kernel, mesh=mesh,
        out_type=jax.ShapeDtypeStruct((NCHUNK, 8, CH), jnp.float32),
        scratch_types=[pltpu.VMEM((9 * NPAD,), jnp.float32),
                       pltpu.VMEM((CH,), jnp.int32),
                       pltpu.VMEM((CH,), jnp.int32),
                       pltpu.VMEM((8, CH), jnp.float32)],
        compiler_params=pltpu.CompilerParams(needs_layout_passes=False),
    )
    def k(att_h, srcp_h, dstp_h, dstwf_h, meta_h, attv, sbuf, dbuf, mbuf):
        wid = lax.axis_index("s") * 2 + lax.axis_index("c")
        pltpu.sync_copy(att_h, attv)
        zero16 = jnp.zeros((16,), jnp.float32)
        for r in range(4, 8):
            for t in range(CH // 16):
                mbuf[r, pl.ds(t * 16, 16)] = zero16
        lo = wid * cpw
        hi = jnp.minimum(lo + cpw, NCHUNK)

        def chunk_body(c, _):
            pltpu.sync_copy(srcp_h.at[pl.ds(c * CH, CH)], sbuf)
            pltpu.sync_copy(dstp_h.at[pl.ds(c * CH, CH)], dbuf)
            pltpu.sync_copy(dstwf_h.at[pl.ds(c * CH, CH)], mbuf.at[3])

            def vec_body(t, _2):
                ids = sbuf[pl.ds(t * 16, 16)]
                idd = dbuf[pl.ds(t * 16, 16)]
                for h in range(3):
                    a = plsc.load_gather(attv, [ids + h * NPAD])
                    b = plsc.load_gather(attv, [idd + (3 + h) * NPAD])
                    l = a + b
                    l = jnp.where(l > 0, l, 0.2 * l)
                    cs = plsc.load_gather(attv, [idd + (6 + h) * NPAD])
                    z = jnp.minimum(l - cs, 70.0)
                    mbuf[h, pl.ds(t * 16, 16)] = jnp.exp(z)
                return _2
            lax.fori_loop(0, CH // 16, vec_body, 0)
            pltpu.sync_copy(mbuf, meta_h.at[c])
            return _

        lax.fori_loop(lo, hi, chunk_body, 0)

    return k(att9, srcp, dstp, dstwf)


def _sc_gather(h, srcp, fin):
    mesh = plsc.VectorSubcoreMesh(core_axis_name="c", subcore_axis_name="s")
    rpw = EPP // 32
    R = 40
    nit = rpw // R

    @functools.partial(
        pl.kernel, mesh=mesh,
        out_type=jax.ShapeDtypeStruct((EPP, fin), jnp.float32),
        scratch_types=[pltpu.VMEM((rpw,), jnp.int32),
                       pltpu.VMEM((R, fin), jnp.float32),
                       pltpu.VMEM((R, fin), jnp.float32),
                       pltpu.SemaphoreType.DMA,
                       pltpu.SemaphoreType.DMA],
        compiler_params=pltpu.CompilerParams(needs_layout_passes=False),
    )
    def k(h_h, srcp_h, g_h, idxv, buf0, buf1, sem0, sem1):
        wid = lax.axis_index("s") * 2 + lax.axis_index("c")
        base = wid * rpw
        pltpu.sync_copy(srcp_h.at[pl.ds(base, rpw)], idxv)

        def start(g, buf, sem):
            pltpu.async_copy(h_h.at[idxv.at[pl.ds(g * R, R)]], buf, sem)

        def wait(buf, sem):
            pltpu.make_async_copy(h_h.at[idxv.at[pl.ds(0, R)]],
                                  buf, sem).wait()

        start(0, buf0, sem0)

        def body(q, _):
            g0 = 2 * q
            g1 = g0 + 1

            @pl.when(g1 < nit)
            def _s1():
                start(g1, buf1, sem1)

            wait(buf0, sem0)
            pltpu.sync_copy(buf0, g_h.at[pl.ds(base + g0 * R, R)])

            @pl.when(g1 < nit)
            def _d1():
                @pl.when(g1 + 1 < nit)
                def _s2():
                    start(g1 + 1, buf0, sem0)

                wait(buf1, sem1)
                pltpu.sync_copy(buf1, g_h.at[pl.ds(base + g1 * R, R)])
            return _

        lax.fori_loop(0, (nit + 1) // 2, body, 0)

    return k(h, srcp)


def _agg_body(cw_ref, cf_ref, meta_ref, g_ref, agg_ref, ssum_ref, *, fin):
    c = pl.program_id(0)
    first = cf_ref[c]
    m = meta_ref[0]
    e3 = m[0:3]
    dstw = m[3]
    dloc = lax.broadcasted_iota(jnp.int32, (WIN, CH), 0).astype(jnp.float32)
    oh = (dstw[None, :] == dloc).astype(jnp.float32)
    A3 = e3[:, None, :] * oh[None, :, :]

    @pl.when(first == 1)
    def _init():
        agg_ref[...] = jnp.zeros((3, WIN, fin), jnp.float32)
        ssum_ref[...] = jnp.zeros((1, 8, WIN), jnp.float32)

    prod = jnp.dot(A3.reshape(3 * WIN, CH), g_ref[...],
                   preferred_element_type=jnp.float32)
    agg_ref[...] += prod.reshape(3, WIN, fin)
    rs = jnp.sum(A3, axis=2)
    ssum_ref[...] += jnp.concatenate(
        [rs, jnp.zeros((5, WIN), jnp.float32)], axis=0)[None]


def _aggregate(cw, cf, meta, G, fin):
    return pl.pallas_call(
        functools.partial(_agg_body, fin=fin),
        grid_spec=pltpu.PrefetchScalarGridSpec(
            num_scalar_prefetch=2,
            grid=(NCHUNK,),
            in_specs=[pl.BlockSpec((1, 8, CH), lambda c, cw, cf: (c, 0, 0)),
                      pl.BlockSpec((CH, fin), lambda c, cw, cf: (c, 0))],
            out_specs=[
                pl.BlockSpec((3, WIN, fin), lambda c, cw, cf: (0, cw[c], 0)),
                pl.BlockSpec((1, 8, WIN), lambda c, cw, cf: (cw[c], 0, 0))],
        ),
        out_shape=[jax.ShapeDtypeStruct((3, NPAD, fin), jnp.float32),
                   jax.ShapeDtypeStruct((NWIN, 8, WIN), jnp.float32)],
    )(cw, cf, meta, G)


def _proj_body(agg_ref, ssum_ref, c_ref, cst_ref, wv_ref, hn_ref, sc_ref, *, fin):
    a = agg_ref[...]
    s = ssum_ref[...][0:3]
    an = a / (s[:, :, None] + 1e-16)
    hn = jnp.broadcast_to(cst_ref[0][None, :], (BM, EMBED))
    for h in range(3):
        hn = hn + jnp.dot(an[h], c_ref[h], preferred_element_type=jnp.float32)
    hn_ref[...] = hn
    sc = jnp.tanh(jnp.dot(hn, wv_ref[0][:, None],
                          preferred_element_type=jnp.float32))
    sc_ref[...] = sc.T


def _project(agg, ssum8, C, cst, wv, fin):
    return pl.pallas_call(
        functools.partial(_proj_body, fin=fin),
        grid=(NPAD // BM,),
        in_specs=[pl.BlockSpec((3, BM, fin), lambda i: (0, i, 0)),
                  pl.BlockSpec((8, BM), lambda i: (0, i)),
                  pl.BlockSpec((3, fin, EMBED), lambda i: (0, 0, 0)),
                  pl.BlockSpec((1, EMBED), lambda i: (0, 0)),
                  pl.BlockSpec((1, EMBED), lambda i: (0, 0))],
        out_specs=[pl.BlockSpec((BM, EMBED), lambda i: (i, 0)),
                   pl.BlockSpec((1, BM), lambda i: (0, i))],
        out_shape=[jax.ShapeDtypeStruct((NPAD, EMBED), jnp.float32),
                   jax.ShapeDtypeStruct((1, NPAD), jnp.float32)],
    )(agg, ssum8, C, cst, wv)


def _topk_body(sc_ref, b_ref, nm_ref, kp_ref, kept_ref):
    i0 = pl.program_id(0) * BM
    si = sc_ref[0, pl.ds(i0, BM)][:, None]
    bi = b_ref[0, pl.ds(i0, BM)][:, None]
    nmi = nm_ref[0, pl.ds(i0, BM)]
    kpi = kp_ref[0, pl.ds(i0, BM)]
    ii = i0 + lax.broadcasted_iota(jnp.int32, (BM, 1), 0).astype(jnp.float32)

    def jstep(j, rank):
        sj = sc_ref[0, pl.ds(j * BM, BM)][None, :]
        bj = b_ref[0, pl.ds(j * BM, BM)][None, :]
        nj = nm_ref[0, pl.ds(j * BM, BM)][None, :]
        jj = (j * BM + lax.broadcasted_iota(jnp.int32, (1, BM), 1).astype(jnp.float32))
        beat = (sj > si) | ((sj == si) & (jj < ii))
        ok = beat & (bj == bi) & (nj > 0)
        return rank + jnp.sum(ok.astype(jnp.float32), axis=1)

    rank = lax.fori_loop(0, NPAD // BM, jstep, jnp.zeros((BM,), jnp.float32))
    kept = (nmi > 0) & (rank < kpi)
    kept_ref[...] = kept.astype(jnp.float32)[None, :]


def _topk(score, batchf, nmaskf, kperb):
    return pl.pallas_call(
        _topk_body,
        grid=(NPAD // BM,),
        in_specs=[pl.BlockSpec((1, NPAD), lambda i: (0, 0))] * 4,
        out_specs=pl.BlockSpec((1, BM), lambda i: (0, i)),
        out_shape=jax.ShapeDtypeStruct((1, NPAD), jnp.float32),
    )(score, batchf, nmaskf, kperb)


def _pool_body(hn_ref, sc_ref, k_ref, b_ref, hsc_ref, ms_ref, mx_ref, cnt_ref):
    i0 = pl.program_id(0) * BM
    sc = sc_ref[0, pl.ds(i0, BM)]
    k = k_ref[0, pl.ds(i0, BM)]
    b = b_ref[0, pl.ds(i0, BM)]
    hs = hn_ref[...] * (sc * k)[:, None]
    hsc_ref[...] = hs
    gio = lax.broadcasted_iota(jnp.int32, (NUM_GRAPHS, BM), 0).astype(jnp.float32)
    oh = (b[None, :] == gio).astype(jnp.float32)

    @pl.when(pl.program_id(0) == 0)
    def _init():
        ms_ref[...] = jnp.zeros((NUM_GRAPHS, EMBED), jnp.float32)
        mx_ref[...] = jnp.full((NUM_GRAPHS, EMBED), -jnp.inf, jnp.float32)
        cnt_ref[...] = jnp.zeros((NUM_GRAPHS, 128), jnp.float32)

    ms_ref[...] += jnp.dot(oh, hs, preferred_element_type=jnp.float32)
    cnt_ref[...] += jnp.dot(oh, jnp.broadcast_to(k[:, None], (BM, 128)),
                            preferred_element_type=jnp.float32)
    for g in range(NUM_GRAPHS):
        self_f = (oh[g] * k)[:, None]
        mg = jnp.max(jnp.where(self_f > 0, hs,
                               jnp.full_like(hs, -jnp.inf)), axis=0)
        mx_ref[g, :] = jnp.maximum(mx_ref[g, :], mg)


def _pool(hn, score, keptf, batchf):
    return pl.pallas_call(
        _pool_body,
        grid=(NPAD // BM,),
        in_specs=[pl.BlockSpec((BM, EMBED), lambda i: (i, 0)),
                  pl.BlockSpec((1, NPAD), lambda i: (0, 0)),
                  pl.BlockSpec((1, NPAD), lambda i: (0, 0)),
                  pl.BlockSpec((1, NPAD), lambda i: (0, 0))],
        out_specs=[pl.BlockSpec((BM, EMBED), lambda i: (i, 0)),
                   pl.BlockSpec((NUM_GRAPHS, EMBED), lambda i: (0, 0)),
                   pl.BlockSpec((NUM_GRAPHS, EMBED), lambda i: (0, 0)),
                   pl.BlockSpec((NUM_GRAPHS, 128), lambda i: (0, 0))],
        out_shape=[jax.ShapeDtypeStruct((NPAD, EMBED), jnp.float32),
                   jax.ShapeDtypeStruct((NUM_GRAPHS, EMBED), jnp.float32),
                   jax.ShapeDtypeStruct((NUM_GRAPHS, EMBED), jnp.float32),
                   jax.ShapeDtypeStruct((NUM_GRAPHS, 128), jnp.float32)],
    )(hn, score, keptf, batchf)


def _final_body(ms1, mx1, c1, ms2, mx2, c2, ms3, mx3, c3,
                w1_ref, b1_ref, w2_ref, b2_ref, o_ref):
    rep = jnp.zeros((NUM_GRAPHS, 2 * EMBED), jnp.float32)
    for ms_ref, mx_ref, c_ref in ((ms1, mx1, c1), (ms2, mx2, c2),
                                  (ms3, mx3, c3)):
        cnt = jnp.maximum(c_ref[...][:, 0:1], 1.0)
        mean = ms_ref[...] / cnt
        mx = mx_ref[...]
        mx = jnp.where(jnp.isfinite(mx), mx, 0.0)
        rep = rep + jnp.concatenate([mean, mx], axis=1)
    t = jnp.dot(rep, w1_ref[...], preferred_element_type=jnp.float32)
    t = jnp.maximum(t + b1_ref[0][None, :], 0.0)
    o = jnp.dot(t, w2_ref[...], preferred_element_type=jnp.float32)
    o_ref[...] = o + b2_ref[0][None, :]


def _final(reps, Wf1, bf1, Wf2p, bf2p):
    args = []
    for ms, mx, c in reps:
        args += [ms, mx, c]
    args += [Wf1, bf1[None, :], Wf2p, bf2p]
    specs = []
    for _ in range(3):
        specs += [pl.BlockSpec((NUM_GRAPHS, EMBED), lambda i: (0, 0)),
                  pl.BlockSpec((NUM_GRAPHS, EMBED), lambda i: (0, 0)),
                  pl.BlockSpec((NUM_GRAPHS, 128), lambda i: (0, 0))]
    specs += [pl.BlockSpec((2 * EMBED, EMBED), lambda i: (0, 0)),
              pl.BlockSpec((1, EMBED), lambda i: (0, 0)),
              pl.BlockSpec((EMBED, 128), lambda i: (0, 0)),
              pl.BlockSpec((1, 128), lambda i: (0, 0))]
    return pl.pallas_call(
        _final_body,
        grid=(1,),
        in_specs=specs,
        out_specs=pl.BlockSpec((NUM_GRAPHS, 128), lambda i: (0, 0)),
        out_shape=jax.ShapeDtypeStruct((NUM_GRAPHS, 128), jnp.float32),
    )(*args)


def kernel(x, edge_att, edge_index, batch_index, W1, as1, ad1, b1, Wl1, bl1, p1,
           W2, as2, ad2, b2, Wl2, bl2, p2, W3, as3, ad3, b3, Wl3, bl3, p3,
           Wf1, bf1, Wf2, bf2):
    N = N_NODES
    params = ((W1, as1, ad1, b1, Wl1, bl1, p1),
              (W2, as2, ad2, b2, Wl2, bl2, p2),
              (W3, as3, ad3, b3, Wl3, bl3, p3))

    loop = jnp.arange(N, dtype=jnp.int32)
    src_all = jnp.concatenate([edge_index[0], loop])
    dst_all = jnp.concatenate([edge_index[1], loop])
    wins = dst_all // WIN
    order = jnp.argsort(wins, stable=True)
    cwnt = jnp.bincount(wins, length=NWIN)
    ncc = jnp.maximum((cwnt + CH - 1) // CH, 1)
    coff = jnp.concatenate([jnp.zeros((1,), ncc.dtype), jnp.cumsum(ncc)[:-1]])
    total = jnp.sum(ncc)
    carr = jnp.arange(NCHUNK, dtype=jnp.int32)
    cw = jnp.clip(jnp.searchsorted(jnp.cumsum(ncc), carr, side='right'),
                  0, NWIN - 1).astype(jnp.int32)
    cf = ((carr == coff[cw]) & (carr < total)).astype(jnp.int32)
    woff = jnp.concatenate([jnp.zeros((1,), cwnt.dtype), jnp.cumsum(cwnt)[:-1]])
    wsorted = wins[order]
    rank = jnp.arange(NE, dtype=jnp.int32) - woff[wsorted].astype(jnp.int32)
    pos = (coff[wsorted].astype(jnp.int32) * CH + rank)
    srcp = jnp.zeros((EPP,), jnp.int32).at[pos].set(
        src_all[order], indices_are_sorted=True, unique_indices=True)
    dstp = jnp.zeros((EPP,), jnp.int32).at[pos].set(
        dst_all[order], indices_are_sorted=True, unique_indices=True)
    dstwf = jnp.full((EPP,), -1.0, jnp.float32).at[pos].set(
        (dst_all[order] - wsorted * WIN).astype(jnp.float32),
        indices_are_sorted=True, unique_indices=True)

    batchp = jnp.concatenate(
        [batch_index, jnp.full((NPAD - N,), NUM_GRAPHS, jnp.int32)])
    batchf = batchp.astype(jnp.float32)[None, :]
    nmaskf = jnp.concatenate(
        [jnp.ones((N,), jnp.float32), jnp.zeros((NPAD - N,), jnp.float32)])[None, :]
    counts0 = jnp.bincount(batch_index, length=NUM_GRAPHS).astype(jnp.float32)

    h = jnp.pad(x, ((0, NPAD - N), (0, 0)))
    keptf = nmaskf
    counts = counts0
    reps = []
    for li in range(3):
        W, a_s, a_d, b, Wl, bl, pw = params[li]
        Fin = W.shape[0]
        Wr = W.reshape(Fin, HEADS, EMBED)
        As = jnp.einsum('fhe,he->fh', Wr, a_s)
        Ad = jnp.einsum('fhe,he->fh', Wr, a_d)
        AsAd = jnp.pad(jnp.concatenate([As, Ad], axis=1), ((0, 0), (0, 122)))
        Wlr = Wl.reshape(HEADS, EMBED, EMBED)
        C = jnp.stack([_mm(Wr[:, hh, :], Wlr[hh]) for hh in range(HEADS)])
        cst = (b @ Wl + bl)[None, :]
        wv = (pw / (jnp.linalg.norm(pw) + 1e-16))[None, :]

        att = _attn_prep(h, AsAd, keptf)
        meta = _sc_meta(att[:9].reshape(-1), srcp, dstp, dstwf)
        G = _sc_gather(h, srcp, Fin)
        agg, ssumw = _aggregate(cw, cf, meta, G, Fin)
        ssum8 = ssumw.transpose(1, 0, 2).reshape(8, NPAD)
        hn, score = _project(agg, ssum8, C, cst, wv, Fin)

        kper = jnp.ceil(RATIO * counts)
        kperb = jnp.concatenate([kper, jnp.zeros((1,), jnp.float32)])[batchp][None, :]
        keptf = _topk(score, batchf, keptf, kperb)
        h, msum, mx, cnt = _pool(hn, score, keptf, batchf)
        counts = cnt[:, 0]
        reps.append((msum, mx, cnt))

    Wf2p = jnp.pad(Wf2, ((0, 0), (0, 126)))
    bf2p = jnp.pad(bf2, (0, 126))[None, :]
    out = _final(reps, Wf1, bf1, Wf2p, bf2p)
    return out[:, :2]
